# Initial kernel scaffold; baseline (speedup 1.0000x reference)
import jax
import jax.numpy as jnp
from jax.experimental import pallas as pl


def kernel(x):
    raise NotImplementedError("write your pallas kernel here")


# speedup vs baseline: 1.0672x; 1.0672x over previous
import functools

import jax
import jax.numpy as jnp
from jax.experimental import pallas as pl
from jax.experimental.pallas import tpu as pltpu

_K0 = 928981903
_K1 = 3453687069

_ROT = (13, 15, 26, 6, 17, 29, 16, 24)


def _threefry_bits(j):
    k0 = jnp.uint32(_K0)
    k1 = jnp.uint32(_K1)
    k2 = jnp.uint32(_K0 ^ _K1 ^ 0x1BD11BDA)
    ks = (k0, k1, k2)

    x0 = jnp.full_like(j, k0)
    x1 = j + k1

    for g in range(5):
        rots = _ROT[0:4] if g % 2 == 0 else _ROT[4:8]
        for r in rots:
            x0 = x0 + x1
            x1 = (x1 << jnp.uint32(r)) | (x1 >> jnp.uint32(32 - r))
            x1 = x1 ^ x0
        x0 = x0 + ks[(g + 1) % 3]
        x1 = x1 + ks[(g + 2) % 3] + jnp.uint32(g + 1)
    return x0 ^ x1


def _binarize_kernel(x_ref, y_ref, *, block_rows, ncols):
    i = pl.program_id(0)
    row = jax.lax.broadcasted_iota(jnp.uint32, x_ref.shape, 0)
    col = jax.lax.broadcasted_iota(jnp.uint32, x_ref.shape, 1)
    j = (jnp.uint32(i * block_rows) + row) * jnp.uint32(ncols) + col
    bits = _threefry_bits(j)
    m = (bits >> jnp.uint32(9)).astype(jnp.int32)
    u_scaled = m.astype(jnp.float32)
    x = x_ref[...]
    thresh = (x + 1.0) * 4194304.0
    y_ref[...] = jnp.where(u_scaled <= thresh, 1.0, -1.0).astype(jnp.float32)


@jax.jit
def kernel(x):
    nrows, ncols = x.shape
    block_rows = 256
    grid = (nrows // block_rows,)
    return pl.pallas_call(
        functools.partial(_binarize_kernel, block_rows=block_rows, ncols=ncols),
        grid=grid,
        in_specs=[pl.BlockSpec((block_rows, ncols), lambda i: (i, 0))],
        out_specs=pl.BlockSpec((block_rows, ncols), lambda i: (i, 0)),
        out_shape=jax.ShapeDtypeStruct((nrows, ncols), jnp.float32),
        compiler_params=pltpu.CompilerParams(
            dimension_semantics=("parallel",),
        ),
    )(x)
